# baseline (speedup 1.0000x reference)
import functools

import jax
import jax.numpy as jnp
from jax import lax
from jax.experimental import pallas as pl
from jax.experimental.pallas import tpu as pltpu
from jax.experimental.pallas import tpu_sc as plsc

_NUM_CORES = 2
_NUM_SUBCORES = 16
_NW = _NUM_CORES * _NUM_SUBCORES
_L = 16
_CHUNK = 128


def _sc_gather_t(coef128, idx):
    B = idx.shape[0]
    D = 32
    b_per_w = B // _NW
    n_groups = b_per_w // _L
    mesh = plsc.VectorSubcoreMesh(core_axis_name="c", subcore_axis_name="s")

    @functools.partial(
        pl.kernel,
        mesh=mesh,
        out_type=jax.ShapeDtypeStruct((D, B), jnp.float32),
        scratch_types=[
            pltpu.VMEM((b_per_w,), jnp.int32),
            pltpu.VMEM((b_per_w,), jnp.int32),
            pltpu.VMEM((_CHUNK, 128), jnp.float32),
            pltpu.VMEM((D, b_per_w), jnp.float32),
            pltpu.SemaphoreType.DMA,
        ],
        compiler_params=pltpu.CompilerParams(needs_layout_passes=False),
    )
    def gather_kernel(table_hbm, idx_hbm, out_hbm, uidx_v, blk_idx_v,
                      blk_v, ct_v, sem):
        wid = lax.axis_index("s") * _NUM_CORES + lax.axis_index("c")
        base = wid * b_per_w
        pltpu.sync_copy(idx_hbm.at[pl.ds(base, b_per_w)], uidx_v)
        for g in range(n_groups):
            u = uidx_v[pl.ds(g * _L, _L)]
            blk_idx_v[pl.ds(g * _L, _L)] = lax.shift_right_logical(u, 2)

        lane = lax.iota(jnp.int32, _L)
        groups_per_chunk = _CHUNK // _L

        def chunk_body(t, carry):
            cbase = t * _CHUNK
            pltpu.async_copy(
                table_hbm.at[blk_idx_v.at[pl.ds(cbase, _CHUNK)]], blk_v, sem
            ).wait()

            def body(g, carry2):
                srow = cbase + g * _L
                rows = lane + g * _L
                u = uidx_v[pl.ds(srow, _L)]
                off = lax.shift_left(jnp.bitwise_and(u, 3), 5)
                for f in range(32):
                    vals = plsc.load_gather(blk_v, [rows, off + f])
                    ct_v[f, pl.ds(srow, _L)] = vals
                return carry2

            lax.fori_loop(0, groups_per_chunk, body, 0)
            return carry

        lax.fori_loop(0, b_per_w // _CHUNK, chunk_body, 0)
        pltpu.sync_copy(ct_v, out_hbm.at[:, pl.ds(base, b_per_w)])

    return gather_kernel(coef128, idx)


def _tc_body(xt_ref, ct_ref, o_ref):
    x = xt_ref[...]
    c = ct_ref[...]
    o_ref[...] = jnp.sum(x * c[None], axis=1)


def _tc_reduce_t(xt, ct):
    I, F, S = xt.shape
    Bs = 1024
    return pl.pallas_call(
        _tc_body,
        grid=(S // Bs,),
        in_specs=[
            pl.BlockSpec((I, F, Bs), lambda i: (0, 0, i)),
            pl.BlockSpec((F, Bs), lambda i: (0, i)),
        ],
        out_specs=pl.BlockSpec((I, Bs), lambda i: (0, i)),
        out_shape=jax.ShapeDtypeStruct((I, S), jnp.float32),
    )(xt, ct)


def kernel(x, user_index, coef):
    v, d = coef.shape
    coef128 = jnp.concatenate(
        [coef[a::4] for a in range(4)], axis=1
    )
    xt = x.transpose(1, 2, 0)
    ct = _sc_gather_t(coef128, user_index.astype(jnp.int32))
    out_t = _tc_reduce_t(xt, ct)
    return out_t.T


# speedup vs baseline: 8.3360x; 8.3360x over previous
import functools

import jax
import jax.numpy as jnp
from jax import lax
from jax.experimental import pallas as pl
from jax.experimental.pallas import tpu as pltpu
from jax.experimental.pallas import tpu_sc as plsc

_NUM_CORES = 2
_NUM_SUBCORES = 16
_NW = _NUM_CORES * _NUM_SUBCORES
_L = 16
_CHUNK = 128


def _sc_gather_t(coef128, idx):
    B = idx.shape[0]
    D = 32
    b_per_w = B // _NW
    n_groups = b_per_w // _L
    mesh = plsc.VectorSubcoreMesh(core_axis_name="c", subcore_axis_name="s")

    @functools.partial(
        pl.kernel,
        mesh=mesh,
        out_type=jax.ShapeDtypeStruct((D, B), jnp.float32),
        scratch_types=[
            pltpu.VMEM((b_per_w,), jnp.int32),
            pltpu.VMEM((b_per_w,), jnp.int32),
            pltpu.VMEM((_CHUNK, 128), jnp.float32),
            pltpu.VMEM((D, b_per_w), jnp.float32),
            pltpu.SemaphoreType.DMA,
        ],
        compiler_params=pltpu.CompilerParams(
            needs_layout_passes=False, skip_device_barrier=True
        ),
    )
    def gather_kernel(table_hbm, idx_hbm, out_hbm, uidx_v, blk_idx_v,
                      blk_v, ct_v, sem):
        wid = lax.axis_index("s") * _NUM_CORES + lax.axis_index("c")
        base = wid * b_per_w
        pltpu.sync_copy(idx_hbm.at[pl.ds(base, b_per_w)], uidx_v)
        for g in range(n_groups):
            u = uidx_v[pl.ds(g * _L, _L)]
            blk_idx_v[pl.ds(g * _L, _L)] = lax.shift_right_logical(u, 2)

        lane = lax.iota(jnp.int32, _L)
        groups_per_chunk = _CHUNK // _L

        def chunk_body(t, carry):
            cbase = t * _CHUNK
            pltpu.async_copy(
                table_hbm.at[blk_idx_v.at[pl.ds(cbase, _CHUNK)]], blk_v, sem
            ).wait()

            def body(g, carry2):
                srow = cbase + g * _L
                rows = lane + g * _L
                u = uidx_v[pl.ds(srow, _L)]
                off = lax.shift_left(jnp.bitwise_and(u, 3), 5)
                for f in range(32):
                    vals = plsc.load_gather(blk_v, [rows, off + f])
                    ct_v[f, pl.ds(srow, _L)] = vals
                return carry2

            lax.fori_loop(0, groups_per_chunk, body, 0)
            return carry

        lax.fori_loop(0, b_per_w // _CHUNK, chunk_body, 0)
        pltpu.sync_copy(ct_v, out_hbm.at[:, pl.ds(base, b_per_w)])

    return gather_kernel(coef128, idx)


def _tc_body(xt_ref, ct_ref, o_ref):
    x = xt_ref[...]
    c = ct_ref[...]
    o_ref[...] = jnp.sum(x * c[None], axis=1)


def _tc_reduce_t(xt, ct):
    I, F, S = xt.shape
    Bs = 1024
    return pl.pallas_call(
        _tc_body,
        grid=(S // Bs,),
        in_specs=[
            pl.BlockSpec((I, F, Bs), lambda i: (0, 0, i)),
            pl.BlockSpec((F, Bs), lambda i: (0, i)),
        ],
        out_specs=pl.BlockSpec((I, Bs), lambda i: (0, i)),
        out_shape=jax.ShapeDtypeStruct((I, S), jnp.float32),
    )(xt, ct)


def kernel(x, user_index, coef):
    v, d = coef.shape
    coef128 = coef.reshape(v * d // 128, 128)
    xt = x.transpose(1, 2, 0)
    ct = _sc_gather_t(coef128, user_index.astype(jnp.int32))
    out_t = _tc_reduce_t(xt, ct)
    return out_t.T
